# baseline (speedup 1.0000x reference)
import jax
import jax.numpy as jnp
from jax import lax
from jax.experimental import pallas as pl
from jax.experimental.pallas import tpu as pltpu
from jax.experimental.pallas import tpu_sc as plsc

_B = 4096
_L = 50
_TT = 257
_MD = 256
_SLOT = 272
_NROWS = _L * _SLOT
_HROWS = _NROWS // 2
_NHIDX = 80
_ZROW = 257


def _pre_body(bd_ref, w_ref, out_ref, wt_ref):
    for j in range(5):
        out_ref[j] = jnp.dot(bd_ref[...], w_ref[j],
                             preferred_element_type=jnp.float32
                             ).astype(jnp.bfloat16)
    wt_ref[...] = w_ref[:, 256:257, :]


def _idx_body(x1_ref, x2_ref, x3_ref, out_ref):
    rows = x1_ref.shape[0]
    hl = _L // 2
    base = lax.broadcasted_iota(jnp.int32, (rows, hl), 1) * _SLOT
    pad = jnp.full((rows, _NHIDX - 3 * hl), _ZROW, jnp.int32)
    for h in range(2):
        s = pl.ds(h * hl, hl)
        out_ref[h] = jnp.concatenate(
            [x1_ref[:, s] + base,
             x2_ref[:, s] + base + 104,
             x3_ref[:, s] + base + 208,
             pad], axis=1)


def _mlp_body(acc_ref, t_ref, wt_ref, b1_ref, w2_ref, b2_ref,
              w3_ref, b3_ref, out_ref):
    h = (acc_ref[0].astype(jnp.float32)
         + acc_ref[1].astype(jnp.float32)
         + jnp.dot(t_ref[...], wt_ref[...],
                   preferred_element_type=jnp.float32)
         + b1_ref[...])
    h = jnp.maximum(h, 0.0)
    h = jnp.maximum(
        jnp.dot(h, w2_ref[...], preferred_element_type=jnp.float32)
        + b2_ref[...], 0.0)
    out_ref[...] = (jnp.dot(h, w3_ref[...],
                            preferred_element_type=jnp.float32)
                    + b3_ref[...])


def _gather_body(table_hbm, idx_hbm, out_hbm, idx_v, buf_v, out_v, table_sh,
                 sem0, sem1):
    sems = (sem0, sem1)
    sid = lax.axis_index("s")
    cid = lax.axis_index("c")

    h0 = pl.multiple_of(cid * _HROWS, 8)

    @pl.when(sid < 15)
    def _():
        r0 = pl.multiple_of(sid * 432, 8)
        pltpu.sync_copy(table_hbm.at[pl.ds(h0 + r0, 432)],
                        table_sh.at[pl.ds(r0, 432)])

    @pl.when(sid == 15)
    def _():
        r0 = pl.multiple_of(h0 + 15 * 432, 8)
        pltpu.sync_copy(table_hbm.at[pl.ds(r0, 320)],
                        table_sh.at[pl.ds(15 * 432, 320)])

    plsc.subcore_barrier()

    def fire(b, slot):
        op = pl.multiple_of(b * _NHIDX, 8)
        pltpu.async_copy(table_sh.at[idx_v.at[pl.ds(op, _NHIDX)]],
                         buf_v.at[slot], sems[slot])

    def wait_slot(slot):
        pltpu.make_async_copy(table_hbm.at[pl.ds(0, _NHIDX)],
                              buf_v.at[slot], sems[slot]).wait()

    def reduce_store(b, slot):
        def rbody(r, accs):
            out = list(accs)
            for u in range(2):
                for j in range(8):
                    pa = (buf_v[slot, 4 * r + 2 * u, pl.ds(32 * j, 32)]
                          + buf_v[slot, 4 * r + 2 * u + 1,
                                  pl.ds(32 * j, 32)])
                    ea, eb = plsc.unpack(pa,
                                         format=plsc.PackFormat.INTERLEAVED)
                    out[2 * j] = out[2 * j] + ea
                    out[2 * j + 1] = out[2 * j + 1] + eb
            return tuple(out)

        accs = lax.fori_loop(
            0, _NHIDX // 4, rbody,
            tuple(jnp.zeros((16,), jnp.float32) for _ in range(16)))
        for j in range(8):
            ob = pl.multiple_of(b * _MD + 32 * j, 8)
            out_v[pl.ds(ob, 32)] = plsc.pack(
                accs[2 * j], accs[2 * j + 1],
                format=plsc.PackFormat.INTERLEAVED)

    for sub in range(4):
        b0 = sid * 256 + sub * 64
        i0 = pl.multiple_of((cid * _B + b0) * _NHIDX, 8)
        pltpu.sync_copy(idx_hbm.at[pl.ds(i0, 64 * _NHIDX)], idx_v)
        fire(0, 0)
        fire(1, 1)

        def pair(bb, carry):
            b = bb * 2
            wait_slot(0)
            reduce_store(b, 0)

            @pl.when(bb < 31)
            def _():
                fire(b + 2, 0)

            wait_slot(1)
            reduce_store(b + 1, 1)

            @pl.when(bb < 31)
            def _():
                fire(b + 3, 1)

            return carry

        lax.fori_loop(0, 32, pair, 0)
        oo = pl.multiple_of((cid * _B + b0) * _MD, 8)
        pltpu.sync_copy(out_v, out_hbm.at[pl.ds(oo, 64 * _MD)])


def _make_gather_sum():
    mesh = plsc.VectorSubcoreMesh(core_axis_name="c", subcore_axis_name="s")
    return pl.kernel(
        _gather_body,
        out_type=jax.ShapeDtypeStruct((2 * _B * _MD,), jnp.bfloat16),
        mesh=mesh,
        scratch_types=[
            pltpu.VMEM((64 * _NHIDX,), jnp.int32),
            pltpu.VMEM((2, _NHIDX, _MD), jnp.bfloat16),
            pltpu.VMEM((64 * _MD,), jnp.bfloat16),
            pltpu.VMEM_SHARED((_HROWS, _MD), jnp.bfloat16),
            pltpu.SemaphoreType.DMA,
            pltpu.SemaphoreType.DMA,
        ],
        compiler_params=pltpu.CompilerParams(use_tc_tiling_on_sc=False,
                                             needs_layout_passes=False),
    )


def kernel(x1, x2, x3, t, mask, device, emb1, emb2, emb3, W1, b1, W2, b2,
           W3, b3):
    del mask, device
    x1 = x1.astype(jnp.int32)
    x2 = x2.astype(jnp.int32)
    x3 = x3.astype(jnp.int32)
    W1r = W1.reshape(_L, _TT, _MD)

    bd = jnp.zeros((_SLOT, _TT), jnp.float32)
    bd = bd.at[0:101, 0:96].set(emb1)
    bd = bd.at[104:205, 96:192].set(emb2)
    bd = bd.at[208:257, 192:256].set(emb3)

    u = pl.pallas_call(
        _pre_body,
        grid=(_L // 5,),
        in_specs=[
            pl.BlockSpec((_SLOT, _TT), lambda l: (0, 0)),
            pl.BlockSpec((5, _TT, _MD), lambda l: (l, 0, 0)),
        ],
        out_specs=[pl.BlockSpec((5, _SLOT, _MD), lambda l: (l, 0, 0)),
                   pl.BlockSpec((5, 1, _MD), lambda l: (l, 0, 0))],
        out_shape=[jax.ShapeDtypeStruct((_L, _SLOT, _MD), jnp.bfloat16),
                   jax.ShapeDtypeStruct((_L, 1, _MD), jnp.float32)],
    )(bd, W1r)
    u, wt = u
    wt = wt.reshape(_L, _MD)
    table = u.reshape(_NROWS, _MD)

    idx = pl.pallas_call(
        _idx_body,
        grid=(_B // 1024,),
        in_specs=[pl.BlockSpec((1024, _L), lambda i: (i, 0))] * 3,
        out_specs=pl.BlockSpec((2, 1024, _NHIDX), lambda i: (0, i, 0)),
        out_shape=jax.ShapeDtypeStruct((2, _B, _NHIDX), jnp.int32),
    )(x1, x2, x3)
    idx_flat = idx.reshape(2 * _B * _NHIDX)

    acc = _make_gather_sum()(table, idx_flat).reshape(2, _B, _MD)

    out = pl.pallas_call(
        _mlp_body,
        grid=(_B // 1024,),
        in_specs=[
            pl.BlockSpec((2, 1024, _MD), lambda i: (0, i, 0)),
            pl.BlockSpec((1024, _L), lambda i: (i, 0)),
            pl.BlockSpec((_L, _MD), lambda i: (0, 0)),
            pl.BlockSpec((1, _MD), lambda i: (0, 0)),
            pl.BlockSpec((_MD, _MD), lambda i: (0, 0)),
            pl.BlockSpec((1, _MD), lambda i: (0, 0)),
            pl.BlockSpec((_MD, 1), lambda i: (0, 0)),
            pl.BlockSpec((1, 1), lambda i: (0, 0)),
        ],
        out_specs=pl.BlockSpec((1024, 1), lambda i: (i, 0)),
        out_shape=jax.ShapeDtypeStruct((_B, 1), jnp.float32),
    )(acc, t, wt, b1.reshape(1, _MD), W2, b2.reshape(1, _MD),
      W3, b3.reshape(1, 1))
    return out


# speedup vs baseline: 1.0134x; 1.0134x over previous
import jax
import jax.numpy as jnp
from jax import lax
from jax.experimental import pallas as pl
from jax.experimental.pallas import tpu as pltpu
from jax.experimental.pallas import tpu_sc as plsc

_B = 4096
_L = 50
_TT = 257
_MD = 256
_SLOT = 272
_NROWS = _L * _SLOT
_HROWS = _NROWS // 2
_NHIDX = 80
_ZROW = 257


def _pre_body(bd_ref, w_ref, out_ref, wt_ref):
    for j in range(5):
        out_ref[j] = jnp.dot(bd_ref[...], w_ref[j],
                             preferred_element_type=jnp.float32
                             ).astype(jnp.bfloat16)
    wt_ref[...] = w_ref[:, 256:257, :]


def _idx_body(x1_ref, x2_ref, x3_ref, out_ref):
    rows = x1_ref.shape[0]
    hl = _L // 2
    base = lax.broadcasted_iota(jnp.int32, (rows, hl), 1) * _SLOT
    pad = jnp.full((rows, _NHIDX - 3 * hl), _ZROW, jnp.int32)
    for h in range(2):
        s = pl.ds(h * hl, hl)
        out_ref[h] = jnp.concatenate(
            [x1_ref[:, s] + base,
             x2_ref[:, s] + base + 104,
             x3_ref[:, s] + base + 208,
             pad], axis=1)


def _mlp_body(acc_ref, t_ref, wt_ref, b1_ref, w2_ref, b2_ref,
              w3_ref, b3_ref, out_ref):
    h = (acc_ref[0].astype(jnp.float32)
         + acc_ref[1].astype(jnp.float32)
         + jnp.dot(t_ref[...], wt_ref[...],
                   preferred_element_type=jnp.float32)
         + b1_ref[...])
    h = jnp.maximum(h, 0.0)
    h = jnp.maximum(
        jnp.dot(h, w2_ref[...], preferred_element_type=jnp.float32)
        + b2_ref[...], 0.0)
    out_ref[...] = (jnp.dot(h, w3_ref[...],
                            preferred_element_type=jnp.float32)
                    + b3_ref[...])


def _gather_body(table_hbm, idx_hbm, out_hbm, idx_v, idx_p, buf_v, out_v,
                 table_sh, sem0, sem1, sem_i, sem_o):
    sems = (sem0, sem1)
    sid = lax.axis_index("s")
    cid = lax.axis_index("c")

    h0 = pl.multiple_of(cid * _HROWS, 8)

    @pl.when(sid < 15)
    def _():
        r0 = pl.multiple_of(sid * 432, 8)
        pltpu.sync_copy(table_hbm.at[pl.ds(h0 + r0, 432)],
                        table_sh.at[pl.ds(r0, 432)])

    @pl.when(sid == 15)
    def _():
        r0 = pl.multiple_of(h0 + 15 * 432, 8)
        pltpu.sync_copy(table_hbm.at[pl.ds(r0, 320)],
                        table_sh.at[pl.ds(15 * 432, 320)])

    plsc.subcore_barrier()

    def fire(b, slot, idxb):
        op = pl.multiple_of(b * _NHIDX, 8)
        pltpu.async_copy(table_sh.at[idxb.at[pl.ds(op, _NHIDX)]],
                         buf_v.at[slot], sems[slot])

    def wait_slot(slot):
        pltpu.make_async_copy(table_hbm.at[pl.ds(0, _NHIDX)],
                              buf_v.at[slot], sems[slot]).wait()

    def reduce_store(b, slot, out_v):
        def rbody(r, accs):
            out = list(accs)
            for u in range(2):
                for j in range(8):
                    pa = (buf_v[slot, 4 * r + 2 * u, pl.ds(32 * j, 32)]
                          + buf_v[slot, 4 * r + 2 * u + 1,
                                  pl.ds(32 * j, 32)])
                    ea, eb = plsc.unpack(pa,
                                         format=plsc.PackFormat.INTERLEAVED)
                    out[2 * j] = out[2 * j] + ea
                    out[2 * j + 1] = out[2 * j + 1] + eb
            return tuple(out)

        accs = lax.fori_loop(
            0, _NHIDX // 4, rbody,
            tuple(jnp.zeros((16,), jnp.float32) for _ in range(16)))
        for j in range(8):
            ob = pl.multiple_of(b * _MD + 32 * j, 8)
            out_v[pl.ds(ob, 32)] = plsc.pack(
                accs[2 * j], accs[2 * j + 1],
                format=plsc.PackFormat.INTERLEAVED)

    def drain_out(osl):
        pltpu.make_async_copy(out_hbm.at[pl.ds(0, 64 * _MD)],
                              out_v.at[osl], sem_o).wait()

    idx_bufs = (idx_v, idx_p)
    i0 = pl.multiple_of((cid * _B + sid * 256) * _NHIDX, 8)
    pltpu.sync_copy(idx_hbm.at[pl.ds(i0, 64 * _NHIDX)], idx_v)
    for sub in range(4):
        b0 = sid * 256 + sub * 64
        osl = sub % 2
        out_s = out_v.at[osl]
        idxb = idx_bufs[sub % 2]
        if sub >= 1:
            pltpu.make_async_copy(idx_hbm.at[pl.ds(0, 64 * _NHIDX)],
                                  idxb, sem_i).wait()
        if sub >= 2:
            drain_out(osl)
        fire(0, 0, idxb)
        fire(1, 1, idxb)
        if sub < 3:
            i1 = pl.multiple_of((cid * _B + b0 + 64) * _NHIDX, 8)
            pltpu.async_copy(idx_hbm.at[pl.ds(i1, 64 * _NHIDX)],
                             idx_bufs[(sub + 1) % 2], sem_i)

        def pair(bb, carry):
            b = bb * 2
            wait_slot(0)
            reduce_store(b, 0, out_s)

            @pl.when(bb < 31)
            def _():
                fire(b + 2, 0, idxb)

            wait_slot(1)
            reduce_store(b + 1, 1, out_s)

            @pl.when(bb < 31)
            def _():
                fire(b + 3, 1, idxb)

            return carry

        lax.fori_loop(0, 32, pair, 0)
        oo = pl.multiple_of((cid * _B + b0) * _MD, 8)
        pltpu.async_copy(out_s, out_hbm.at[pl.ds(oo, 64 * _MD)], sem_o)
    drain_out(0)
    drain_out(1)


def _make_gather_sum():
    mesh = plsc.VectorSubcoreMesh(core_axis_name="c", subcore_axis_name="s")
    return pl.kernel(
        _gather_body,
        out_type=jax.ShapeDtypeStruct((2 * _B * _MD,), jnp.bfloat16),
        mesh=mesh,
        scratch_types=[
            pltpu.VMEM((64 * _NHIDX,), jnp.int32),
            pltpu.VMEM((64 * _NHIDX,), jnp.int32),
            pltpu.VMEM((2, _NHIDX, _MD), jnp.bfloat16),
            pltpu.VMEM((2, 64 * _MD), jnp.bfloat16),
            pltpu.VMEM_SHARED((_HROWS, _MD), jnp.bfloat16),
            pltpu.SemaphoreType.DMA,
            pltpu.SemaphoreType.DMA,
            pltpu.SemaphoreType.DMA,
            pltpu.SemaphoreType.DMA,
        ],
        compiler_params=pltpu.CompilerParams(use_tc_tiling_on_sc=False,
                                             needs_layout_passes=False),
    )


def kernel(x1, x2, x3, t, mask, device, emb1, emb2, emb3, W1, b1, W2, b2,
           W3, b3):
    del mask, device
    x1 = x1.astype(jnp.int32)
    x2 = x2.astype(jnp.int32)
    x3 = x3.astype(jnp.int32)
    W1r = W1.reshape(_L, _TT, _MD)

    bd = jnp.zeros((_SLOT, _TT), jnp.float32)
    bd = bd.at[0:101, 0:96].set(emb1)
    bd = bd.at[104:205, 96:192].set(emb2)
    bd = bd.at[208:257, 192:256].set(emb3)

    u = pl.pallas_call(
        _pre_body,
        grid=(_L // 5,),
        in_specs=[
            pl.BlockSpec((_SLOT, _TT), lambda l: (0, 0)),
            pl.BlockSpec((5, _TT, _MD), lambda l: (l, 0, 0)),
        ],
        out_specs=[pl.BlockSpec((5, _SLOT, _MD), lambda l: (l, 0, 0)),
                   pl.BlockSpec((5, 1, _MD), lambda l: (l, 0, 0))],
        out_shape=[jax.ShapeDtypeStruct((_L, _SLOT, _MD), jnp.bfloat16),
                   jax.ShapeDtypeStruct((_L, 1, _MD), jnp.float32)],
    )(bd, W1r)
    u, wt = u
    wt = wt.reshape(_L, _MD)
    table = u.reshape(_NROWS, _MD)

    idx = pl.pallas_call(
        _idx_body,
        grid=(_B // 1024,),
        in_specs=[pl.BlockSpec((1024, _L), lambda i: (i, 0))] * 3,
        out_specs=pl.BlockSpec((2, 1024, _NHIDX), lambda i: (0, i, 0)),
        out_shape=jax.ShapeDtypeStruct((2, _B, _NHIDX), jnp.int32),
    )(x1, x2, x3)
    idx_flat = idx.reshape(2 * _B * _NHIDX)

    acc = _make_gather_sum()(table, idx_flat).reshape(2, _B, _MD)

    out = pl.pallas_call(
        _mlp_body,
        grid=(_B // 1024,),
        in_specs=[
            pl.BlockSpec((2, 1024, _MD), lambda i: (0, i, 0)),
            pl.BlockSpec((1024, _L), lambda i: (i, 0)),
            pl.BlockSpec((_L, _MD), lambda i: (0, 0)),
            pl.BlockSpec((1, _MD), lambda i: (0, 0)),
            pl.BlockSpec((_MD, _MD), lambda i: (0, 0)),
            pl.BlockSpec((1, _MD), lambda i: (0, 0)),
            pl.BlockSpec((_MD, 1), lambda i: (0, 0)),
            pl.BlockSpec((1, 1), lambda i: (0, 0)),
        ],
        out_specs=pl.BlockSpec((1024, 1), lambda i: (i, 0)),
        out_shape=jax.ShapeDtypeStruct((_B, 1), jnp.float32),
    )(acc, t, wt, b1.reshape(1, _MD), W2, b2.reshape(1, _MD),
      W3, b3.reshape(1, 1))
    return out
